# baseline (speedup 1.0000x reference)
import functools

import jax
import jax.numpy as jnp
from jax import lax
from jax.experimental import pallas as pl
from jax.experimental.pallas import tpu as pltpu

NEG_SLOPE = 0.2
_ROW_TILE = 256


def _leaky(y):
    return jnp.where(y >= 0, y, NEG_SLOPE * y)


def _encode_gram_kernel(t_ref, w_ref, b_ref, m_ref, rs_ref, *, bz, R):
    t2 = t_ref[...].reshape(bz * R, -1)
    x = jnp.dot(t2, w_ref[...],
                preferred_element_type=jnp.float32) + b_ref[...]
    x = x - jnp.max(x, axis=-1, keepdims=True)
    e = jnp.exp(x)
    p = e / jnp.sum(e, axis=-1, keepdims=True)
    for b in range(bz):
        pb = p[b * R:(b + 1) * R, :]
        mb = lax.dot_general(pb, pb,
                             dimension_numbers=(((1,), (1,)), ((), ())),
                             preferred_element_type=jnp.float32)
        m_ref[b] = mb
        rs_ref[pl.ds(b * R, R), :] = jnp.sum(mb, axis=-1, keepdims=True)


def _encode_softmax_gram(t, enc_w, enc_b, bz, R):
    T = t.shape[-1]
    rows = bz * R
    E = enc_w.shape[1]
    kern = functools.partial(_encode_gram_kernel, bz=bz, R=R)
    return pl.pallas_call(
        kern,
        out_shape=(jax.ShapeDtypeStruct((bz, R, R), jnp.float32),
                   jax.ShapeDtypeStruct((rows, 1), jnp.float32)),
        grid=(1,),
        in_specs=[pl.BlockSpec((bz, R, T), lambda i: (0, 0, 0)),
                  pl.BlockSpec((T, E), lambda i: (0, 0)),
                  pl.BlockSpec((1, E), lambda i: (0, 0))],
        out_specs=[pl.BlockSpec((bz, R, R), lambda i: (0, 0, 0)),
                   pl.BlockSpec((rows, 1), lambda i: (0, 0))],
    )(t, enc_w, enc_b.reshape(1, E))


def _block1_kernel(ps_ref, nd_ref, rs_ref, fw_ref, fb_ref,
                   w0_ref, b0_ref, w1_ref, b1_ref, o_ref):
    ps = _leaky(jnp.dot(ps_ref[...], fw_ref[...],
                        preferred_element_type=jnp.float32) + fb_ref[...])
    node = nd_ref[...] + ps
    h = rs_ref[...] * node
    h = _leaky(jnp.dot(h, w0_ref[...],
                       preferred_element_type=jnp.float32) + b0_ref[...])
    o_ref[...] = jnp.dot(h, w1_ref[...],
                         preferred_element_type=jnp.float32) + b1_ref[...]


def _block2_kernel(x_ref, sc_ref, sh_ref, rs_ref, w_ref, b_ref, o_ref):
    xn = x_ref[...] * sc_ref[...] + sh_ref[...]
    h = rs_ref[...] * xn
    o_ref[...] = _leaky(jnp.dot(h, w_ref[...],
                                preferred_element_type=jnp.float32) + b_ref[...])


def _block3_kernel(x_ref, sc_ref, sh_ref, rs_ref,
                   w0_ref, b0_ref, w1_ref, b1_ref, o_ref):
    xn = x_ref[...] * sc_ref[...] + sh_ref[...]
    h = rs_ref[...] * xn
    h = _leaky(jnp.dot(h, w0_ref[...],
                       preferred_element_type=jnp.float32) + b0_ref[...])
    o_ref[...] = _leaky(jnp.dot(h, w1_ref[...],
                                preferred_element_type=jnp.float32) + b1_ref[...])


def _row_fused_call(kern_fn, inputs, tiled, rows, dout):
    tm = rows if rows <= _ROW_TILE else _ROW_TILE
    in_specs = []
    for a, is_tiled in zip(inputs, tiled):
        c = a.shape[1]
        if is_tiled:
            in_specs.append(pl.BlockSpec((tm, c), lambda i: (i, 0)))
        else:
            in_specs.append(pl.BlockSpec((a.shape[0], c), lambda i: (0, 0)))
    return pl.pallas_call(
        kern_fn,
        out_shape=jax.ShapeDtypeStruct((rows, dout), jnp.float32),
        grid=(pl.cdiv(rows, tm),),
        in_specs=in_specs,
        out_specs=pl.BlockSpec((tm, dout), lambda i: (i, 0)),
        compiler_params=pltpu.CompilerParams(
            dimension_semantics=("parallel",)),
    )(*inputs)


def _mstats_kernel(m_ref, v_ref, *, bz, R):
    n = float(R * R)
    acc = None
    for b in range(bz):
        mb = m_ref[b]
        s = jnp.sum(mb, keepdims=True)
        q = jnp.sum(mb * mb, keepdims=True)
        var = (q - s * s * (1.0 / n)) * (1.0 / (n - 1.0))
        acc = var if acc is None else acc + var
    v_ref[...] = acc * (1.0 / bz)


def _edge_variance(m, bz, R):
    v = pl.pallas_call(
        functools.partial(_mstats_kernel, bz=bz, R=R),
        out_shape=jax.ShapeDtypeStruct((1, 1), jnp.float32),
        grid=(1,),
        in_specs=[pl.BlockSpec((bz, R, R), lambda i: (0, 0, 0))],
        out_specs=pl.BlockSpec((1, 1), lambda i: (0, 0)),
    )(m)
    return v.reshape(())


def _bn_scale_shift(x2d, gamma, beta, eps=1e-5):
    mean = jnp.mean(x2d, axis=0)
    var = jnp.mean(jnp.square(x2d - mean), axis=0)
    scale = gamma / jnp.sqrt(var + eps)
    shift = beta - mean * scale
    return scale.reshape(1, -1), shift.reshape(1, -1)


def _batchnorm1d_3d(x, gamma, beta, eps=1e-5):
    mean = jnp.mean(x, axis=(0, 2), keepdims=True)
    var = jnp.mean(jnp.square(x - mean), axis=(0, 2), keepdims=True)
    return (x - mean) / jnp.sqrt(var + eps) * gamma[None, :, None] + beta[None, :, None]


def _tail_kernel(sc_ref, x_ref, w0e_ref, b0_ref,
                 w1_ref, b1_ref, w2_ref, b2_ref, o_ref, *, R, l, nb):
    scg = sc_ref[...]
    scgT = jnp.transpose(scg)
    iota_lane = lax.broadcasted_iota(jnp.int32, (R, R), 1)
    iota_sub = lax.broadcasted_iota(jnp.int32, (R, R), 0)
    for b in range(nb):
        sc_lane = scg[b:b + 1, :]
        sc_sub = scgT[:, b:b + 1]
        less = sc_lane < sc_sub
        tie = (sc_lane == sc_sub) & (iota_lane < iota_sub)
        rank = jnp.sum((less | tie).astype(jnp.int32), axis=1, keepdims=True)
        sel = (rank == iota_lane) & (iota_sub < l)
        xp = jnp.dot(sel.astype(jnp.float32), x_ref[b],
                     preferred_element_type=jnp.float32,
                     precision=lax.Precision.HIGHEST)
        xq = xp[:, 0:1] * w0e_ref[0]
        for e in range(1, 8):
            xq += xp[:, e:e + 1] * w0e_ref[e]
        h = _leaky(jnp.sum(xq, axis=0, keepdims=True) + b0_ref[...])
        h = _leaky(jnp.dot(h, w1_ref[...],
                           preferred_element_type=jnp.float32) + b1_ref[...])
        o_ref[b:b + 1, :] = jnp.dot(
            h, w2_ref[...], preferred_element_type=jnp.float32) + b2_ref[...]


def _select_mlp(score, x, fcn_w0, fcn_b0, fcn_w1, fcn_b1, fcn_w2, fcn_b2):
    bz, R = score.shape
    l = int(R * 0.7)
    nb = bz // 2
    w0e = jnp.pad(fcn_w0.reshape(l, 8, -1),
                  ((0, R - l), (0, 0), (0, 0))).transpose(1, 0, 2)
    dh = fcn_w0.shape[1]
    kern = functools.partial(_tail_kernel, R=R, l=l, nb=nb)
    return pl.pallas_call(
        kern,
        out_shape=jax.ShapeDtypeStruct((bz, 2), jnp.float32),
        grid=(2,),
        in_specs=[pl.BlockSpec((nb, R), lambda i: (i, 0)),
                  pl.BlockSpec((nb, R, 8), lambda i: (i, 0, 0)),
                  pl.BlockSpec((8, R, dh), lambda i: (0, 0, 0)),
                  pl.BlockSpec((1, dh), lambda i: (0, 0)),
                  pl.BlockSpec((dh, 32), lambda i: (0, 0)),
                  pl.BlockSpec((1, 32), lambda i: (0, 0)),
                  pl.BlockSpec((32, 2), lambda i: (0, 0)),
                  pl.BlockSpec((1, 2), lambda i: (0, 0))],
        out_specs=pl.BlockSpec((nb, 2), lambda i: (i, 0)),
        compiler_params=pltpu.CompilerParams(
            dimension_semantics=("parallel",)),
    )(score, x, w0e,
      fcn_b0.reshape(1, -1), fcn_w1, fcn_b1.reshape(1, -1),
      fcn_w2, fcn_b2.reshape(1, -1))


def kernel(enc_w, enc_b, fcp_w, fcp_b, gcn_w0, gcn_b0, gcn_w1, gcn_b1,
           gcn1_w, gcn1_b, gcn2_w0, gcn2_b0, gcn2_w1, gcn2_b1, score_w,
           fcn_w0, fcn_b0, fcn_w1, fcn_b1, fcn_w2, fcn_b2,
           bn1_g, bn1_b, bn2_g, bn2_b, bn3_g, bn3_b,
           t, nodes, pseudo):
    bz, R, T = t.shape
    rows = bz * R
    D = nodes.shape[-1]

    m, rs = _encode_softmax_gram(t, enc_w, enc_b, bz, R)

    edge_variance = _edge_variance(m, bz, R)

    x1 = _row_fused_call(
        _block1_kernel,
        [pseudo.reshape(rows, R), nodes.reshape(rows, D), rs,
         fcp_w, fcp_b.reshape(1, -1),
         gcn_w0, gcn_b0.reshape(1, -1),
         gcn_w1, gcn_b1.reshape(1, -1)],
        [True, True, True, False, False, False, False, False, False],
        rows, R)

    sc1, sh1 = _bn_scale_shift(x1, bn1_g, bn1_b)
    x2 = _row_fused_call(
        _block2_kernel,
        [x1, sc1, sh1, rs, gcn1_w, gcn1_b.reshape(1, -1)],
        [True, False, False, True, False, False],
        rows, R)

    sc2, sh2 = _bn_scale_shift(x2, bn2_g, bn2_b)
    x3 = _row_fused_call(
        _block3_kernel,
        [x2, sc2, sh2, rs,
         gcn2_w0, gcn2_b0.reshape(1, -1),
         gcn2_w1, gcn2_b1.reshape(1, -1)],
        [True, False, False, True, False, False, False, False],
        rows, 8)

    x = _batchnorm1d_3d(x3.reshape(bz, R, 8), bn3_g, bn3_b)

    score = jax.nn.sigmoid(jnp.sum(x * score_w, axis=-1))
    sc = score
    logits = _select_mlp(score, x, fcn_w0, fcn_b0, fcn_w1, fcn_b1,
                         fcn_w2, fcn_b2)

    return (logits, sc), m, edge_variance


# speedup vs baseline: 1.0475x; 1.0475x over previous
import functools

import jax
import jax.numpy as jnp
from jax import lax
from jax.experimental import pallas as pl
from jax.experimental.pallas import tpu as pltpu

NEG_SLOPE = 0.2
_ROW_TILE = 256


def _leaky(y):
    return jnp.where(y >= 0, y, NEG_SLOPE * y)


def _encode_gram_kernel(t_ref, w_ref, b_ref, m_ref, rs_ref, *, bz, R):
    t2 = t_ref[...].reshape(bz * R, -1)
    x = jnp.dot(t2, w_ref[...],
                preferred_element_type=jnp.float32) + b_ref[...]
    x = x - jnp.max(x, axis=-1, keepdims=True)
    e = jnp.exp(x)
    p = e / jnp.sum(e, axis=-1, keepdims=True)
    for b in range(bz):
        pb = p[b * R:(b + 1) * R, :]
        mb = lax.dot_general(pb, pb,
                             dimension_numbers=(((1,), (1,)), ((), ())),
                             preferred_element_type=jnp.float32)
        m_ref[b] = mb
        rs_ref[pl.ds(b * R, R), :] = jnp.sum(mb, axis=-1, keepdims=True)


def _encode_softmax_gram(t, enc_w, enc_b, bz, R):
    T = t.shape[-1]
    rows = bz * R
    E = enc_w.shape[1]
    kern = functools.partial(_encode_gram_kernel, bz=bz, R=R)
    return pl.pallas_call(
        kern,
        out_shape=(jax.ShapeDtypeStruct((bz, R, R), jnp.float32),
                   jax.ShapeDtypeStruct((rows, 1), jnp.float32)),
        grid=(1,),
        in_specs=[pl.BlockSpec((bz, R, T), lambda i: (0, 0, 0)),
                  pl.BlockSpec((T, E), lambda i: (0, 0)),
                  pl.BlockSpec((1, E), lambda i: (0, 0))],
        out_specs=[pl.BlockSpec((bz, R, R), lambda i: (0, 0, 0)),
                   pl.BlockSpec((rows, 1), lambda i: (0, 0))],
    )(t, enc_w, enc_b.reshape(1, E))


def _block1_kernel(ps_ref, nd_ref, rs_ref, fw_ref, fb_ref,
                   w0_ref, b0_ref, w1_ref, b1_ref, o_ref):
    ps = _leaky(jnp.dot(ps_ref[...], fw_ref[...],
                        preferred_element_type=jnp.float32) + fb_ref[...])
    node = nd_ref[...] + ps
    h = rs_ref[...] * node
    h = _leaky(jnp.dot(h, w0_ref[...],
                       preferred_element_type=jnp.float32) + b0_ref[...])
    o_ref[...] = jnp.dot(h, w1_ref[...],
                         preferred_element_type=jnp.float32) + b1_ref[...]


def _block2_kernel(x_ref, sc_ref, sh_ref, rs_ref, w_ref, b_ref, o_ref):
    xn = x_ref[...] * sc_ref[...] + sh_ref[...]
    h = rs_ref[...] * xn
    o_ref[...] = _leaky(jnp.dot(h, w_ref[...],
                                preferred_element_type=jnp.float32) + b_ref[...])


def _block3_kernel(x_ref, sc_ref, sh_ref, rs_ref,
                   w0_ref, b0_ref, w1_ref, b1_ref, o_ref):
    xn = x_ref[...] * sc_ref[...] + sh_ref[...]
    h = rs_ref[...] * xn
    h = _leaky(jnp.dot(h, w0_ref[...],
                       preferred_element_type=jnp.float32) + b0_ref[...])
    o_ref[...] = _leaky(jnp.dot(h, w1_ref[...],
                                preferred_element_type=jnp.float32) + b1_ref[...])


def _row_fused_call(kern_fn, inputs, tiled, rows, dout):
    tm = rows if rows <= _ROW_TILE else _ROW_TILE
    in_specs = []
    for a, is_tiled in zip(inputs, tiled):
        c = a.shape[1]
        if is_tiled:
            in_specs.append(pl.BlockSpec((tm, c), lambda i: (i, 0)))
        else:
            in_specs.append(pl.BlockSpec((a.shape[0], c), lambda i: (0, 0)))
    return pl.pallas_call(
        kern_fn,
        out_shape=jax.ShapeDtypeStruct((rows, dout), jnp.float32),
        grid=(pl.cdiv(rows, tm),),
        in_specs=in_specs,
        out_specs=pl.BlockSpec((tm, dout), lambda i: (i, 0)),
        compiler_params=pltpu.CompilerParams(
            dimension_semantics=("parallel",)),
    )(*inputs)


def _mstats_kernel(m_ref, v_ref, *, bz, R):
    n = float(R * R)
    acc = None
    for b in range(bz):
        mb = m_ref[b]
        s = jnp.sum(mb, keepdims=True)
        q = jnp.sum(mb * mb, keepdims=True)
        var = (q - s * s * (1.0 / n)) * (1.0 / (n - 1.0))
        acc = var if acc is None else acc + var
    v_ref[...] = acc * (1.0 / bz)


def _edge_variance(m, bz, R):
    v = pl.pallas_call(
        functools.partial(_mstats_kernel, bz=bz, R=R),
        out_shape=jax.ShapeDtypeStruct((1, 1), jnp.float32),
        grid=(1,),
        in_specs=[pl.BlockSpec((bz, R, R), lambda i: (0, 0, 0))],
        out_specs=pl.BlockSpec((1, 1), lambda i: (0, 0)),
    )(m)
    return v.reshape(())


def _bn_scale_shift(x2d, gamma, beta, eps=1e-5):
    mean = jnp.mean(x2d, axis=0)
    var = jnp.mean(jnp.square(x2d - mean), axis=0)
    scale = gamma / jnp.sqrt(var + eps)
    shift = beta - mean * scale
    return scale.reshape(1, -1), shift.reshape(1, -1)


def _batchnorm1d_3d(x, gamma, beta, eps=1e-5):
    mean = jnp.mean(x, axis=(0, 2), keepdims=True)
    var = jnp.mean(jnp.square(x - mean), axis=(0, 2), keepdims=True)
    return (x - mean) / jnp.sqrt(var + eps) * gamma[None, :, None] + beta[None, :, None]


def _tail_kernel(sc_ref, x_ref, w0e_ref, b0_ref,
                 w1_ref, b1_ref, w2_ref, b2_ref, o_ref, *, R, l, nb):
    scg = sc_ref[...]
    scgT = jnp.transpose(scg)
    iota_lane = lax.broadcasted_iota(jnp.int32, (R, R), 1)
    iota_sub = lax.broadcasted_iota(jnp.int32, (R, R), 0)
    for b in range(nb):
        sc_lane = scg[b:b + 1, :]
        sc_sub = scgT[:, b:b + 1]
        less = sc_lane < sc_sub
        tie = (sc_lane == sc_sub) & (iota_lane < iota_sub)
        rank = jnp.sum((less | tie).astype(jnp.int32), axis=1, keepdims=True)
        sel = (rank == iota_lane) & (iota_sub < l)
        xp = jnp.dot(sel.astype(jnp.float32), x_ref[b],
                     preferred_element_type=jnp.float32)
        xq = xp[:, 0:1] * w0e_ref[0]
        for e in range(1, 8):
            xq += xp[:, e:e + 1] * w0e_ref[e]
        h = _leaky(jnp.sum(xq, axis=0, keepdims=True) + b0_ref[...])
        h = _leaky(jnp.dot(h, w1_ref[...],
                           preferred_element_type=jnp.float32) + b1_ref[...])
        o_ref[b:b + 1, :] = jnp.dot(
            h, w2_ref[...], preferred_element_type=jnp.float32) + b2_ref[...]


def _select_mlp(score, x, fcn_w0, fcn_b0, fcn_w1, fcn_b1, fcn_w2, fcn_b2):
    bz, R = score.shape
    l = int(R * 0.7)
    nb = bz // 2
    w0e = jnp.pad(fcn_w0.reshape(l, 8, -1),
                  ((0, R - l), (0, 0), (0, 0))).transpose(1, 0, 2)
    dh = fcn_w0.shape[1]
    kern = functools.partial(_tail_kernel, R=R, l=l, nb=nb)
    return pl.pallas_call(
        kern,
        out_shape=jax.ShapeDtypeStruct((bz, 2), jnp.float32),
        grid=(2,),
        in_specs=[pl.BlockSpec((nb, R), lambda i: (i, 0)),
                  pl.BlockSpec((nb, R, 8), lambda i: (i, 0, 0)),
                  pl.BlockSpec((8, R, dh), lambda i: (0, 0, 0)),
                  pl.BlockSpec((1, dh), lambda i: (0, 0)),
                  pl.BlockSpec((dh, 32), lambda i: (0, 0)),
                  pl.BlockSpec((1, 32), lambda i: (0, 0)),
                  pl.BlockSpec((32, 2), lambda i: (0, 0)),
                  pl.BlockSpec((1, 2), lambda i: (0, 0))],
        out_specs=pl.BlockSpec((nb, 2), lambda i: (i, 0)),
        compiler_params=pltpu.CompilerParams(
            dimension_semantics=("parallel",)),
    )(score, x, w0e,
      fcn_b0.reshape(1, -1), fcn_w1, fcn_b1.reshape(1, -1),
      fcn_w2, fcn_b2.reshape(1, -1))


def kernel(enc_w, enc_b, fcp_w, fcp_b, gcn_w0, gcn_b0, gcn_w1, gcn_b1,
           gcn1_w, gcn1_b, gcn2_w0, gcn2_b0, gcn2_w1, gcn2_b1, score_w,
           fcn_w0, fcn_b0, fcn_w1, fcn_b1, fcn_w2, fcn_b2,
           bn1_g, bn1_b, bn2_g, bn2_b, bn3_g, bn3_b,
           t, nodes, pseudo):
    bz, R, T = t.shape
    rows = bz * R
    D = nodes.shape[-1]

    m, rs = _encode_softmax_gram(t, enc_w, enc_b, bz, R)

    edge_variance = _edge_variance(m, bz, R)

    x1 = _row_fused_call(
        _block1_kernel,
        [pseudo.reshape(rows, R), nodes.reshape(rows, D), rs,
         fcp_w, fcp_b.reshape(1, -1),
         gcn_w0, gcn_b0.reshape(1, -1),
         gcn_w1, gcn_b1.reshape(1, -1)],
        [True, True, True, False, False, False, False, False, False],
        rows, R)

    sc1, sh1 = _bn_scale_shift(x1, bn1_g, bn1_b)
    x2 = _row_fused_call(
        _block2_kernel,
        [x1, sc1, sh1, rs, gcn1_w, gcn1_b.reshape(1, -1)],
        [True, False, False, True, False, False],
        rows, R)

    sc2, sh2 = _bn_scale_shift(x2, bn2_g, bn2_b)
    x3 = _row_fused_call(
        _block3_kernel,
        [x2, sc2, sh2, rs,
         gcn2_w0, gcn2_b0.reshape(1, -1),
         gcn2_w1, gcn2_b1.reshape(1, -1)],
        [True, False, False, True, False, False, False, False],
        rows, 8)

    x = _batchnorm1d_3d(x3.reshape(bz, R, 8), bn3_g, bn3_b)

    score = jax.nn.sigmoid(jnp.sum(x * score_w, axis=-1))
    sc = score
    logits = _select_mlp(score, x, fcn_w0, fcn_b0, fcn_w1, fcn_b1,
                         fcn_w2, fcn_b2)

    return (logits, sc), m, edge_variance
